# Initial kernel scaffold; baseline (speedup 1.0000x reference)
import jax
import jax.numpy as jnp
from jax.experimental import pallas as pl


def kernel(context, target, mask, adj_ei, transit_ei, W1, b1, g1, be1, W2, b2, Wt, bt, mask_token, Wg1, bg1, Wg2, bg2, Wtg1, btg1, Wtg2, btg2, alpha, Wh1, bh1, gh, beh, Wh2, bh2):
    raise NotImplementedError("write your pallas kernel here")


# speedup vs baseline: 7.7990x; 7.7990x over previous
import functools
import math

import jax
import jax.numpy as jnp
from jax import lax
from jax.experimental import pallas as pl
from jax.experimental.pallas import tpu as pltpu
from jax.experimental.pallas import tpu_sc as plsc

N = 50000
E = 800000
H = 64
QW = 16
HH = 32
CTX = 128
TGT = 16

NC = 2
NS = 16
CH = 128
NCHUNK = 392
NBLK = 2
BCH = NCHUNK // NBLK
EPT = NCHUNK * CH
E_PAD = EPT * NS
ACC_ROWS = 50176
ROWS_PT = ACC_ROWS // NS
ZROWS = 112
ZCH = ROWS_PT // ZROWS
DW = 8
JUNK = N

ROWT = 2000
GRID = N // ROWT


def _gelu(x):
    return 0.5 * x * (1.0 + lax.erf(x * (1.0 / math.sqrt(2.0))))


def _ln(x, g, b):
    m = jnp.mean(x, axis=-1, keepdims=True)
    v = jnp.mean((x - m) ** 2, axis=-1, keepdims=True)
    return (x - m) * lax.rsqrt(v + 1e-5) * g + b


def _sc_mesh():
    return plsc.VectorSubcoreMesh(core_axis_name="c", subcore_axis_name="s",
                                  num_cores=NC, num_subcores=NS)


def _sc_conv(tab, srcs, dsts, zeros_h):
    @functools.partial(
        pl.kernel,
        out_type=jax.ShapeDtypeStruct((NC, ACC_ROWS, QW), jnp.float32),
        mesh=_sc_mesh(),
        scratch_types=[
            pltpu.VMEM((BCH, CH), jnp.int32),
            pltpu.VMEM((BCH, CH), jnp.int32),
            pltpu.VMEM((CH, QW), jnp.float32),
            pltpu.VMEM((CH, QW), jnp.float32),
            pltpu.VMEM((ZROWS, QW), jnp.float32),
            pltpu.VMEM((8, QW), jnp.float32),
            pltpu.VMEM_SHARED((ACC_ROWS, QW), jnp.float32),
            pltpu.SemaphoreType.DMA,
            pltpu.SemaphoreType.DMA,
        ],
        compiler_params=pltpu.CompilerParams(use_tc_tiling_on_sc=False,
                                             needs_layout_passes=False),
    )
    def conv_kernel(tab_hbm, src_hbm, dst_hbm, zeros_hbm, out_hbm,
                    src_v, dst_v, buf0, buf1, zero_v, tag_v, acc, sem0, sem1):
        c = lax.axis_index("c")
        s = lax.axis_index("s")
        pltpu.sync_copy(tab_hbm.at[pl.ds(4 * N, 8)], tag_v)
        p = jnp.max(tag_v[0, :]).astype(jnp.int32)
        srcp = src_hbm.at[p].at[c].at[s]
        dstp = dst_hbm.at[p].at[c].at[s]
        pltpu.sync_copy(zeros_hbm, zero_v)

        def zc(k, carry):
            pltpu.sync_copy(zero_v,
                            acc.at[pl.ds(s * ROWS_PT + k * ZROWS, ZROWS)])
            return carry
        lax.fori_loop(0, ZCH, zc, 0)
        plsc.subcore_barrier()

        def block(b, carry):
            pltpu.sync_copy(srcp.at[pl.ds(b * BCH, BCH)], src_v)
            pltpu.sync_copy(dstp.at[pl.ds(b * BCH, BCH)], dst_v)
            pltpu.async_copy(tab_hbm.at[src_v.at[0]], buf0, sem0)

            def pair(j, carry2):
                e0 = 2 * j
                e1 = e0 + 1
                pltpu.async_copy(tab_hbm.at[src_v.at[e1]], buf1, sem1)
                pltpu.make_async_copy(tab_hbm.at[src_v.at[e0]],
                                      buf0, sem0).wait()
                pltpu.sync_copy(buf0, acc.at[dst_v.at[e0]], add=True)

                @pl.when(j + 1 < BCH // 2)
                def _():
                    pltpu.async_copy(tab_hbm.at[src_v.at[e0 + 2]], buf0, sem0)

                pltpu.make_async_copy(tab_hbm.at[src_v.at[e1]],
                                      buf1, sem1).wait()
                pltpu.sync_copy(buf1, acc.at[dst_v.at[e1]], add=True)
                return carry2
            lax.fori_loop(0, BCH // 2, pair, 0)
            return carry
        lax.fori_loop(0, NBLK, block, 0)
        plsc.subcore_barrier()
        pltpu.sync_copy(acc.at[pl.ds(s * ROWS_PT, ROWS_PT)],
                        out_hbm.at[c].at[pl.ds(s * ROWS_PT, ROWS_PT)])

    return conv_kernel(tab, srcs, dsts, zeros_h)


def _row_spec(w):
    return pl.BlockSpec((ROWT, w), lambda i: (i, 0))


def _full_spec(shape):
    nd = len(shape)
    return pl.BlockSpec(shape, lambda i: (0,) * nd)


def _tc_encoder(context, target, mf,
                W1, b1, g1, be1, W2, b2, Wt, bt, mask_token, Wg1, Wtg1):
    def body(ctx_r, tgt_r, mf_r, W1r, b1r, g1r, be1r,
             W2r, b2r, Wtr, btr, mtokr, Wg1r, Wtg1r,
             fused_r, ha_r, ht_r):
        x = jnp.dot(ctx_r[...], W1r[...],
                    preferred_element_type=jnp.float32) + b1r[...]
        x = _gelu(_ln(x, g1r[...], be1r[...]))
        x = _gelu(jnp.dot(x, W2r[...],
                          preferred_element_type=jnp.float32) + b2r[...])
        m = mf_r[...]
        mt = tgt_r[...] * (1.0 - m) + mtokr[...] * m
        t = _gelu(jnp.dot(mt, Wtr[...],
                          preferred_element_type=jnp.float32) + btr[...])
        fused = jnp.concatenate([x, t], axis=1)
        fused_r[...] = fused
        ha_r[...] = jnp.dot(fused, Wg1r[...],
                            preferred_element_type=jnp.float32)
        ht_r[...] = jnp.dot(fused, Wtg1r[...],
                            preferred_element_type=jnp.float32)

    return pl.pallas_call(
        body,
        grid=(GRID,),
        in_specs=[
            _row_spec(CTX), _row_spec(TGT), _row_spec(TGT),
            _full_spec((CTX, H)), _full_spec((1, H)), _full_spec((1, H)),
            _full_spec((1, H)), _full_spec((H, H)), _full_spec((1, H)),
            _full_spec((TGT, HH)), _full_spec((1, HH)), _full_spec((1, TGT)),
            _full_spec((H + HH, H)), _full_spec((H + HH, H)),
        ],
        out_specs=[_row_spec(H + HH), _row_spec(H), _row_spec(H)],
        out_shape=[
            jax.ShapeDtypeStruct((N, H + HH), jnp.float32),
            jax.ShapeDtypeStruct((N, H), jnp.float32),
            jax.ShapeDtypeStruct((N, H), jnp.float32),
        ],
    )(context, target, mf, W1, b1, g1, be1, W2, b2, Wt, bt,
      mask_token, Wg1, Wtg1)


def _tc_scale(dega, degt, ha, ht):
    def body(dega_r, degt_r, ha_r, ht_r, ga_r, gt_r):
        da = lax.rsqrt(dega_r[...][:, :1] + 1.0)
        dt = lax.rsqrt(degt_r[...][:, :1] + 1.0)
        ga_r[...] = ha_r[...] * da
        gt_r[...] = ht_r[...] * dt

    return pl.pallas_call(
        body,
        grid=(GRID,),
        in_specs=[_row_spec(DW), _row_spec(DW), _row_spec(H), _row_spec(H)],
        out_specs=[_row_spec(H), _row_spec(H)],
        out_shape=[
            jax.ShapeDtypeStruct((N, H), jnp.float32),
            jax.ShapeDtypeStruct((N, H), jnp.float32),
        ],
    )(dega, degt, ha, ht)


def _tc_mid(S, g, deg, bg, W):
    def body(S_r, g_r, deg_r, bgr, Wr, out_r):
        d = lax.rsqrt(deg_r[...][:, :1] + 1.0)
        h = _gelu(d * (S_r[...] + g_r[...]) + bgr[...])
        out_r[...] = jnp.dot(h, Wr[...],
                             preferred_element_type=jnp.float32) * d

    return pl.pallas_call(
        body,
        grid=(GRID,),
        in_specs=[
            _row_spec(H), _row_spec(H), _row_spec(DW),
            _full_spec((1, H)), _full_spec((H, H)),
        ],
        out_specs=_row_spec(H),
        out_shape=jax.ShapeDtypeStruct((N, H), jnp.float32),
    )(S, g, deg, bg, W)


def _tc_head(S2a, S2t, g2a, g2t, dega, degt, fused, alpha,
             bg2, btg2, Wh1, bh1, gh, beh, Wh2, bh2):
    def body(S2a_r, S2t_r, g2a_r, g2t_r, dega_r, degt_r, fused_r, al_r,
             bg2r, btg2r, Wh1r, bh1r, ghr, behr, Wh2r, bh2r, out_r):
        da = lax.rsqrt(dega_r[...][:, :1] + 1.0)
        dt = lax.rsqrt(degt_r[...][:, :1] + 1.0)
        h_sp = da * (S2a_r[...] + g2a_r[...]) + bg2r[...]
        h_tr = dt * (S2t_r[...] + g2t_r[...]) + btg2r[...]
        a = jax.nn.sigmoid(al_r[...])
        h = a * h_sp + (1.0 - a) * h_tr
        z = jnp.concatenate([h, fused_r[...]], axis=1)
        z = jnp.dot(z, Wh1r[...], preferred_element_type=jnp.float32)
        z = _gelu(_ln(z + bh1r[...], ghr[...], behr[...]))
        out_r[...] = jnp.dot(z, Wh2r[...],
                             preferred_element_type=jnp.float32) + bh2r[...]

    D2 = H + H + HH
    return pl.pallas_call(
        body,
        grid=(GRID,),
        in_specs=[
            _row_spec(H), _row_spec(H), _row_spec(H), _row_spec(H),
            _row_spec(DW), _row_spec(DW), _row_spec(H + HH),
            _full_spec((1, 1)),
            _full_spec((1, H)), _full_spec((1, H)),
            _full_spec((D2, H)), _full_spec((1, H)),
            _full_spec((1, H)), _full_spec((1, H)),
            _full_spec((H, TGT)), _full_spec((1, TGT)),
        ],
        out_specs=_row_spec(TGT),
        out_shape=jax.ShapeDtypeStruct((N, TGT), jnp.float32),
    )(S2a, S2t, g2a, g2t, dega, degt, fused, alpha,
      bg2, btg2, Wh1, bh1, gh, beh, Wh2, bh2)


def _tile_layout(idx):
    return idx.reshape(NS, NCHUNK, CH)


def _prep_edges(ei):
    src = ei[0].astype(jnp.int32)
    dst = ei[1].astype(jnp.int32)
    pad = E_PAD - E
    srcp = jnp.concatenate([src, jnp.zeros((pad,), jnp.int32)])
    dstp = jnp.concatenate([dst, jnp.full((pad,), JUNK, jnp.int32)])
    return srcp, dstp


def _quarters(g):
    return jnp.concatenate(
        [g[:, 0:QW], g[:, QW:2 * QW], g[:, 2 * QW:3 * QW], g[:, 3 * QW:4 * QW]],
        axis=0)


def _merge2(S):
    return jnp.concatenate([S[0, :N], S[1, :N]], axis=1)


def kernel(context, target, mask, adj_ei, transit_ei, W1, b1, g1, be1, W2, b2,
           Wt, bt, mask_token, Wg1, bg1, Wg2, bg2, Wtg1, btg1, Wtg2, btg2,
           alpha, Wh1, bh1, gh, beh, Wh2, bh2):
    f32 = jnp.float32
    mf = mask.astype(f32)
    r = lambda v: v.reshape(1, -1).astype(f32)

    src_a, dst_a = _prep_edges(adj_ei)
    src_t, dst_t = _prep_edges(transit_ei)

    la, lt = _tile_layout(dst_a), _tile_layout(dst_t)
    dd = jnp.stack([jnp.stack([la, lt])])
    da = jnp.stack([la, la])[None]
    dt = jnp.stack([lt, lt])[None]

    def offs(src, p):
        return jnp.stack([_tile_layout(src + (2 * p) * N),
                          _tile_layout(src + (2 * p + 1) * N)])

    sa_lo, sa_hi = offs(src_a, 0)[None], offs(src_a, 1)[None]
    st_lo, st_hi = offs(src_t, 0)[None], offs(src_t, 1)[None]
    sd = jnp.stack([jnp.stack([_tile_layout(dst_a),
                               _tile_layout(dst_t + N)])])

    srcs_xs = jnp.concatenate(
        [sd, sa_lo, sa_hi, st_lo, st_hi, sa_lo, sa_hi, st_lo, st_hi])
    dsts_xs = jnp.concatenate([dd, da, da, dt, dt, da, da, dt, dt])

    zeros_c = jnp.zeros((ZROWS, QW), f32)
    fused, ha, ht = _tc_encoder(
        context, target, mf, W1, r(b1), r(g1), r(be1), W2, r(b2), Wt, r(bt),
        mask_token.reshape(1, TGT), Wg1, Wtg1)

    zN = jnp.zeros((N, H), f32)
    init = dict(
        next_tab=jnp.ones((4 * N, QW), f32),
        pend_tab=jnp.zeros((4 * N, QW), f32),
        Slo=jnp.zeros((N, 2 * QW), f32),
        ga=zN, gt=zN, g2a=zN, g2t=zN, S2a=zN, S2t=zN,
        dega=jnp.ones((N, DW), f32), degt=jnp.ones((N, DW), f32),
    )

    def b_deg(cr, S):
        dega = S[0, :N, :DW]
        degt = S[1, :N, :DW]
        ga, gt = _tc_scale(dega, degt, ha, ht)
        return dict(cr, next_tab=_quarters(ga), pend_tab=_quarters(gt),
                    ga=ga, gt=gt, dega=dega, degt=degt)

    def b_lo(cr, S):
        return dict(cr, Slo=_merge2(S))

    def b_c1a(cr, S):
        S1a = jnp.concatenate([cr["Slo"], _merge2(S)], axis=1)
        g2a = _tc_mid(S1a, cr["ga"], cr["dega"], r(bg1), Wg2)
        return dict(cr, next_tab=cr["pend_tab"], pend_tab=_quarters(g2a),
                    g2a=g2a)

    def b_c1t(cr, S):
        S1t = jnp.concatenate([cr["Slo"], _merge2(S)], axis=1)
        g2t = _tc_mid(S1t, cr["gt"], cr["degt"], r(btg1), Wtg2)
        return dict(cr, next_tab=cr["pend_tab"], pend_tab=_quarters(g2t),
                    g2t=g2t)

    def b_c2a(cr, S):
        S2a = jnp.concatenate([cr["Slo"], _merge2(S)], axis=1)
        return dict(cr, next_tab=cr["pend_tab"], S2a=S2a)

    def b_c2t(cr, S):
        S2t = jnp.concatenate([cr["Slo"], _merge2(S)], axis=1)
        return dict(cr, S2t=S2t)

    n_pass = lax.optimization_barrier(jnp.int32(9))

    def cond(st):
        i, _ = st
        return i < n_pass

    def body(st):
        i, cr = st
        tag = jnp.full((8, QW), 1.0, f32) * i.astype(f32)
        tab = jnp.concatenate([cr["next_tab"], tag], axis=0)
        S = _sc_conv(tab, srcs_xs, dsts_xs, zeros_c)
        cr = lax.switch(i, [b_deg, b_lo, b_c1a, b_lo, b_c1t,
                           b_lo, b_c2a, b_lo, b_c2t], cr, S)
        return (i + 1, cr)

    _, fin = lax.while_loop(cond, body, (jnp.int32(0), init))

    return _tc_head(fin["S2a"], fin["S2t"], fin["g2a"], fin["g2t"],
                    fin["dega"], fin["degt"], fused, alpha.reshape(1, 1),
                    r(bg2), r(btg2), Wh1, r(bh1), r(gh), r(beh), Wh2, r(bh2))


# speedup vs baseline: 8.1145x; 1.0405x over previous
import functools
import math

import jax
import jax.numpy as jnp
from jax import lax
from jax.experimental import pallas as pl
from jax.experimental.pallas import tpu as pltpu
from jax.experimental.pallas import tpu_sc as plsc

N = 50000
E = 800000
H = 64
QW = 16
HH = 32
CTX = 128
TGT = 16

NC = 2
NS = 16
CH = 128
NCHUNK = 392
NBLK = 2
BCH = NCHUNK // NBLK
EPT = NCHUNK * CH
E_PAD = EPT * NS
ACC_ROWS = 50176
ROWS_PT = ACC_ROWS // NS
ZROWS = 112
ZCH = ROWS_PT // ZROWS
DW = 8
JUNK = N

ROWT = 2000
GRID = N // ROWT


def _gelu(x):
    return 0.5 * x * (1.0 + lax.erf(x * (1.0 / math.sqrt(2.0))))


def _ln(x, g, b):
    m = jnp.mean(x, axis=-1, keepdims=True)
    v = jnp.mean((x - m) ** 2, axis=-1, keepdims=True)
    return (x - m) * lax.rsqrt(v + 1e-5) * g + b


def _sc_mesh():
    return plsc.VectorSubcoreMesh(core_axis_name="c", subcore_axis_name="s",
                                  num_cores=NC, num_subcores=NS)


def _sc_conv(tab, srcs, dsts, zeros_h):
    @functools.partial(
        pl.kernel,
        out_type=jax.ShapeDtypeStruct((NC, ACC_ROWS, QW), jnp.float32),
        mesh=_sc_mesh(),
        scratch_types=[
            pltpu.VMEM((BCH, CH), jnp.int32),
            pltpu.VMEM((BCH, CH), jnp.int32),
            pltpu.VMEM((CH, QW), jnp.float32),
            pltpu.VMEM((CH, QW), jnp.float32),
            pltpu.VMEM((CH, QW), jnp.float32),
            pltpu.VMEM((CH, QW), jnp.float32),
            pltpu.VMEM((ZROWS, QW), jnp.float32),
            pltpu.VMEM((8, QW), jnp.float32),
            pltpu.VMEM_SHARED((ACC_ROWS, QW), jnp.float32),
            pltpu.SemaphoreType.DMA,
            pltpu.SemaphoreType.DMA,
            pltpu.SemaphoreType.DMA,
            pltpu.SemaphoreType.DMA,
            pltpu.SemaphoreType.DMA,
            pltpu.SemaphoreType.DMA,
            pltpu.SemaphoreType.DMA,
            pltpu.SemaphoreType.DMA,
            pltpu.SemaphoreType.DMA,
        ],
        compiler_params=pltpu.CompilerParams(use_tc_tiling_on_sc=False,
                                             needs_layout_passes=False),
    )
    def conv_kernel(tab_hbm, src_hbm, dst_hbm, zeros_hbm, out_hbm,
                    src_v, dst_v, buf0, buf1, buf2, buf3, zero_v, tag_v, acc,
                    g0, g1, g2, g3, s0, s1, s2, s3, zsem):
        c = lax.axis_index("c")
        s = lax.axis_index("s")
        bufs = [buf0, buf1, buf2, buf3]
        gsem = [g0, g1, g2, g3]
        ssem = [s0, s1, s2, s3]
        pltpu.sync_copy(tab_hbm.at[pl.ds(4 * N, 8)], tag_v)
        p = jnp.max(tag_v[0, :]).astype(jnp.int32)
        srcp = src_hbm.at[p].at[c].at[s]
        dstp = dst_hbm.at[p].at[c].at[s]
        pltpu.sync_copy(zeros_hbm, zero_v)

        def zc(k, carry):
            pltpu.make_async_copy(
                zero_v, acc.at[pl.ds(s * ROWS_PT + k * ZROWS, ZROWS)],
                zsem).start()
            return carry
        lax.fori_loop(0, ZCH, zc, 0)

        def zw(k, carry):
            pltpu.make_async_copy(
                zero_v, acc.at[pl.ds(s * ROWS_PT + k * ZROWS, ZROWS)],
                zsem).wait()
            return carry
        lax.fori_loop(0, ZCH, zw, 0)
        plsc.subcore_barrier()

        def gath(j, b):
            return pltpu.make_async_copy(tab_hbm.at[src_v.at[j]],
                                         bufs[b], gsem[b])

        def scat(j, b):
            return pltpu.make_async_copy(bufs[b],
                                         acc.at[dst_v.at[j]], ssem[b])

        def block(b, carry):
            pltpu.sync_copy(srcp.at[pl.ds(b * BCH, BCH)], src_v)
            pltpu.sync_copy(dstp.at[pl.ds(b * BCH, BCH)], dst_v)
            gath(0, 0).start()
            gath(1, 1).start()

            def quad(q, carry2):
                for k in range(4):
                    j = 4 * q + k
                    gath(j, k).wait()
                    scat(j, k).start(add=True)

                    @pl.when(j >= 2)
                    def _():
                        scat(j - 2, (k + 2) % 4).wait()

                    @pl.when(j + 2 < BCH)
                    def _():
                        gath(j + 2, (k + 2) % 4).start()
                return carry2
            lax.fori_loop(0, BCH // 4, quad, 0)
            scat(BCH - 2, 2).wait()
            scat(BCH - 1, 3).wait()
            return carry
        lax.fori_loop(0, NBLK, block, 0)
        plsc.subcore_barrier()
        pltpu.sync_copy(acc.at[pl.ds(s * ROWS_PT, ROWS_PT)],
                        out_hbm.at[c].at[pl.ds(s * ROWS_PT, ROWS_PT)])

    return conv_kernel(tab, srcs, dsts, zeros_h)


def _row_spec(w):
    return pl.BlockSpec((ROWT, w), lambda i: (i, 0))


def _full_spec(shape):
    nd = len(shape)
    return pl.BlockSpec(shape, lambda i: (0,) * nd)


def _tc_encoder(context, target, mf,
                W1, b1, g1, be1, W2, b2, Wt, bt, mask_token, Wg1, Wtg1):
    def body(ctx_r, tgt_r, mf_r, W1r, b1r, g1r, be1r,
             W2r, b2r, Wtr, btr, mtokr, Wg1r, Wtg1r,
             fused_r, ha_r, ht_r):
        x = jnp.dot(ctx_r[...], W1r[...],
                    preferred_element_type=jnp.float32) + b1r[...]
        x = _gelu(_ln(x, g1r[...], be1r[...]))
        x = _gelu(jnp.dot(x, W2r[...],
                          preferred_element_type=jnp.float32) + b2r[...])
        m = mf_r[...]
        mt = tgt_r[...] * (1.0 - m) + mtokr[...] * m
        t = _gelu(jnp.dot(mt, Wtr[...],
                          preferred_element_type=jnp.float32) + btr[...])
        fused = jnp.concatenate([x, t], axis=1)
        fused_r[...] = fused
        ha_r[...] = jnp.dot(fused, Wg1r[...],
                            preferred_element_type=jnp.float32)
        ht_r[...] = jnp.dot(fused, Wtg1r[...],
                            preferred_element_type=jnp.float32)

    return pl.pallas_call(
        body,
        grid=(GRID,),
        in_specs=[
            _row_spec(CTX), _row_spec(TGT), _row_spec(TGT),
            _full_spec((CTX, H)), _full_spec((1, H)), _full_spec((1, H)),
            _full_spec((1, H)), _full_spec((H, H)), _full_spec((1, H)),
            _full_spec((TGT, HH)), _full_spec((1, HH)), _full_spec((1, TGT)),
            _full_spec((H + HH, H)), _full_spec((H + HH, H)),
        ],
        out_specs=[_row_spec(H + HH), _row_spec(H), _row_spec(H)],
        out_shape=[
            jax.ShapeDtypeStruct((N, H + HH), jnp.float32),
            jax.ShapeDtypeStruct((N, H), jnp.float32),
            jax.ShapeDtypeStruct((N, H), jnp.float32),
        ],
    )(context, target, mf, W1, b1, g1, be1, W2, b2, Wt, bt,
      mask_token, Wg1, Wtg1)


def _tc_scale(dega, degt, ha, ht):
    def body(dega_r, degt_r, ha_r, ht_r, ga_r, gt_r):
        da = lax.rsqrt(dega_r[...][:, :1] + 1.0)
        dt = lax.rsqrt(degt_r[...][:, :1] + 1.0)
        ga_r[...] = ha_r[...] * da
        gt_r[...] = ht_r[...] * dt

    return pl.pallas_call(
        body,
        grid=(GRID,),
        in_specs=[_row_spec(DW), _row_spec(DW), _row_spec(H), _row_spec(H)],
        out_specs=[_row_spec(H), _row_spec(H)],
        out_shape=[
            jax.ShapeDtypeStruct((N, H), jnp.float32),
            jax.ShapeDtypeStruct((N, H), jnp.float32),
        ],
    )(dega, degt, ha, ht)


def _tc_mid(S, g, deg, bg, W):
    def body(S_r, g_r, deg_r, bgr, Wr, out_r):
        d = lax.rsqrt(deg_r[...][:, :1] + 1.0)
        h = _gelu(d * (S_r[...] + g_r[...]) + bgr[...])
        out_r[...] = jnp.dot(h, Wr[...],
                             preferred_element_type=jnp.float32) * d

    return pl.pallas_call(
        body,
        grid=(GRID,),
        in_specs=[
            _row_spec(H), _row_spec(H), _row_spec(DW),
            _full_spec((1, H)), _full_spec((H, H)),
        ],
        out_specs=_row_spec(H),
        out_shape=jax.ShapeDtypeStruct((N, H), jnp.float32),
    )(S, g, deg, bg, W)


def _tc_head(S2a, S2t, g2a, g2t, dega, degt, fused, alpha,
             bg2, btg2, Wh1, bh1, gh, beh, Wh2, bh2):
    def body(S2a_r, S2t_r, g2a_r, g2t_r, dega_r, degt_r, fused_r, al_r,
             bg2r, btg2r, Wh1r, bh1r, ghr, behr, Wh2r, bh2r, out_r):
        da = lax.rsqrt(dega_r[...][:, :1] + 1.0)
        dt = lax.rsqrt(degt_r[...][:, :1] + 1.0)
        h_sp = da * (S2a_r[...] + g2a_r[...]) + bg2r[...]
        h_tr = dt * (S2t_r[...] + g2t_r[...]) + btg2r[...]
        a = jax.nn.sigmoid(al_r[...])
        h = a * h_sp + (1.0 - a) * h_tr
        z = jnp.concatenate([h, fused_r[...]], axis=1)
        z = jnp.dot(z, Wh1r[...], preferred_element_type=jnp.float32)
        z = _gelu(_ln(z + bh1r[...], ghr[...], behr[...]))
        out_r[...] = jnp.dot(z, Wh2r[...],
                             preferred_element_type=jnp.float32) + bh2r[...]

    D2 = H + H + HH
    return pl.pallas_call(
        body,
        grid=(GRID,),
        in_specs=[
            _row_spec(H), _row_spec(H), _row_spec(H), _row_spec(H),
            _row_spec(DW), _row_spec(DW), _row_spec(H + HH),
            _full_spec((1, 1)),
            _full_spec((1, H)), _full_spec((1, H)),
            _full_spec((D2, H)), _full_spec((1, H)),
            _full_spec((1, H)), _full_spec((1, H)),
            _full_spec((H, TGT)), _full_spec((1, TGT)),
        ],
        out_specs=_row_spec(TGT),
        out_shape=jax.ShapeDtypeStruct((N, TGT), jnp.float32),
    )(S2a, S2t, g2a, g2t, dega, degt, fused, alpha,
      bg2, btg2, Wh1, bh1, gh, beh, Wh2, bh2)


def _tile_layout(idx):
    return idx.reshape(NS, NCHUNK, CH)


def _prep_edges(ei):
    src = ei[0].astype(jnp.int32)
    dst = ei[1].astype(jnp.int32)
    pad = E_PAD - E
    srcp = jnp.concatenate([src, jnp.zeros((pad,), jnp.int32)])
    dstp = jnp.concatenate([dst, jnp.full((pad,), JUNK, jnp.int32)])
    return srcp, dstp


def _quarters(g):
    return jnp.concatenate(
        [g[:, 0:QW], g[:, QW:2 * QW], g[:, 2 * QW:3 * QW], g[:, 3 * QW:4 * QW]],
        axis=0)


def _merge2(S):
    return jnp.concatenate([S[0, :N], S[1, :N]], axis=1)


def kernel(context, target, mask, adj_ei, transit_ei, W1, b1, g1, be1, W2, b2,
           Wt, bt, mask_token, Wg1, bg1, Wg2, bg2, Wtg1, btg1, Wtg2, btg2,
           alpha, Wh1, bh1, gh, beh, Wh2, bh2):
    f32 = jnp.float32
    mf = mask.astype(f32)
    r = lambda v: v.reshape(1, -1).astype(f32)

    src_a, dst_a = _prep_edges(adj_ei)
    src_t, dst_t = _prep_edges(transit_ei)

    la, lt = _tile_layout(dst_a), _tile_layout(dst_t)
    dd = jnp.stack([jnp.stack([la, lt])])
    da = jnp.stack([la, la])[None]
    dt = jnp.stack([lt, lt])[None]

    def offs(src, p):
        return jnp.stack([_tile_layout(src + (2 * p) * N),
                          _tile_layout(src + (2 * p + 1) * N)])

    sa_lo, sa_hi = offs(src_a, 0)[None], offs(src_a, 1)[None]
    st_lo, st_hi = offs(src_t, 0)[None], offs(src_t, 1)[None]
    sd = jnp.stack([jnp.stack([_tile_layout(dst_a),
                               _tile_layout(dst_t + N)])])

    srcs_xs = jnp.concatenate(
        [sd, sa_lo, sa_hi, st_lo, st_hi, sa_lo, sa_hi, st_lo, st_hi])
    dsts_xs = jnp.concatenate([dd, da, da, dt, dt, da, da, dt, dt])

    zeros_c = jnp.zeros((ZROWS, QW), f32)
    fused, ha, ht = _tc_encoder(
        context, target, mf, W1, r(b1), r(g1), r(be1), W2, r(b2), Wt, r(bt),
        mask_token.reshape(1, TGT), Wg1, Wtg1)

    zN = jnp.zeros((N, H), f32)
    init = dict(
        next_tab=jnp.ones((4 * N, QW), f32),
        pend_tab=jnp.zeros((4 * N, QW), f32),
        Slo=jnp.zeros((N, 2 * QW), f32),
        ga=zN, gt=zN, g2a=zN, g2t=zN, S2a=zN, S2t=zN,
        dega=jnp.ones((N, DW), f32), degt=jnp.ones((N, DW), f32),
    )

    def b_deg(cr, S):
        dega = S[0, :N, :DW]
        degt = S[1, :N, :DW]
        ga, gt = _tc_scale(dega, degt, ha, ht)
        return dict(cr, next_tab=_quarters(ga), pend_tab=_quarters(gt),
                    ga=ga, gt=gt, dega=dega, degt=degt)

    def b_lo(cr, S):
        return dict(cr, Slo=_merge2(S))

    def b_c1a(cr, S):
        S1a = jnp.concatenate([cr["Slo"], _merge2(S)], axis=1)
        g2a = _tc_mid(S1a, cr["ga"], cr["dega"], r(bg1), Wg2)
        return dict(cr, next_tab=cr["pend_tab"], pend_tab=_quarters(g2a),
                    g2a=g2a)

    def b_c1t(cr, S):
        S1t = jnp.concatenate([cr["Slo"], _merge2(S)], axis=1)
        g2t = _tc_mid(S1t, cr["gt"], cr["degt"], r(btg1), Wtg2)
        return dict(cr, next_tab=cr["pend_tab"], pend_tab=_quarters(g2t),
                    g2t=g2t)

    def b_c2a(cr, S):
        S2a = jnp.concatenate([cr["Slo"], _merge2(S)], axis=1)
        return dict(cr, next_tab=cr["pend_tab"], S2a=S2a)

    def b_c2t(cr, S):
        S2t = jnp.concatenate([cr["Slo"], _merge2(S)], axis=1)
        return dict(cr, S2t=S2t)

    n_pass = lax.optimization_barrier(jnp.int32(9))

    def cond(st):
        i, _ = st
        return i < n_pass

    def body(st):
        i, cr = st
        tag = jnp.full((8, QW), 1.0, f32) * i.astype(f32)
        tab = jnp.concatenate([cr["next_tab"], tag], axis=0)
        S = _sc_conv(tab, srcs_xs, dsts_xs, zeros_c)
        cr = lax.switch(i, [b_deg, b_lo, b_c1a, b_lo, b_c1t,
                           b_lo, b_c2a, b_lo, b_c2t], cr, S)
        return (i + 1, cr)

    _, fin = lax.while_loop(cond, body, (jnp.int32(0), init))

    return _tc_head(fin["S2a"], fin["S2t"], fin["g2a"], fin["g2t"],
                    fin["dega"], fin["degt"], fused, alpha.reshape(1, 1),
                    r(bg2), r(btg2), Wh1, r(bh1), r(gh), r(beh), Wh2, r(bh2))


# speedup vs baseline: 9.7751x; 1.2047x over previous
import functools
import math

import jax
import jax.numpy as jnp
from jax import lax
from jax.experimental import pallas as pl
from jax.experimental.pallas import tpu as pltpu
from jax.experimental.pallas import tpu_sc as plsc

N = 50000
E = 800000
H = 64
QW = 16
HH = 32
CTX = 128
TGT = 16

NC = 2
NS = 16
CH = 128
NCHUNK = 392
NBLK = 2
BCH = NCHUNK // NBLK
EPT = NCHUNK * CH
E_PAD = EPT * NS
ACC_ROWS = 50176
ROWS_PT = ACC_ROWS // NS
ZROWS = 112
ZCH = ROWS_PT // ZROWS
DW = 8
JUNK = N

ROWT = 2000
GRID = N // ROWT


def _gelu(x):
    return 0.5 * x * (1.0 + lax.erf(x * (1.0 / math.sqrt(2.0))))


def _ln(x, g, b):
    m = jnp.mean(x, axis=-1, keepdims=True)
    v = jnp.mean((x - m) ** 2, axis=-1, keepdims=True)
    return (x - m) * lax.rsqrt(v + 1e-5) * g + b


def _sc_mesh():
    return plsc.VectorSubcoreMesh(core_axis_name="c", subcore_axis_name="s",
                                  num_cores=NC, num_subcores=NS)


def _sc_conv(tab, srcs, dsts, zeros_h):
    @functools.partial(
        pl.kernel,
        out_type=jax.ShapeDtypeStruct((2, NC, ACC_ROWS, QW), jnp.float32),
        mesh=_sc_mesh(),
        scratch_types=[
            pltpu.VMEM((BCH, CH), jnp.int32),
            pltpu.VMEM((BCH, CH), jnp.int32),
            pltpu.VMEM((CH, QW), jnp.float32),
            pltpu.VMEM((CH, QW), jnp.float32),
            pltpu.VMEM((CH, QW), jnp.float32),
            pltpu.VMEM((CH, QW), jnp.float32),
            pltpu.VMEM((ZROWS, QW), jnp.float32),
            pltpu.VMEM((8, QW), jnp.float32),
            pltpu.VMEM_SHARED((ACC_ROWS, QW), jnp.float32),
            pltpu.SemaphoreType.DMA,
            pltpu.SemaphoreType.DMA,
            pltpu.SemaphoreType.DMA,
            pltpu.SemaphoreType.DMA,
            pltpu.SemaphoreType.DMA,
            pltpu.SemaphoreType.DMA,
            pltpu.SemaphoreType.DMA,
            pltpu.SemaphoreType.DMA,
            pltpu.SemaphoreType.DMA,
        ],
        compiler_params=pltpu.CompilerParams(use_tc_tiling_on_sc=False,
                                             needs_layout_passes=False),
    )
    def conv_kernel(tab_hbm, src_hbm, dst_hbm, zeros_hbm, out_hbm,
                    src_v, dst_v, buf0, buf1, buf2, buf3, zero_v, tag_v, acc,
                    g0, g1, g2, g3, s0, s1, s2, s3, zsem):
        c = lax.axis_index("c")
        s = lax.axis_index("s")
        bufs = [buf0, buf1, buf2, buf3]
        gsem = [g0, g1, g2, g3]
        ssem = [s0, s1, s2, s3]
        pltpu.sync_copy(tab_hbm.at[pl.ds(4 * N, 8)], tag_v)
        p = jnp.max(tag_v[0, :]).astype(jnp.int32)
        dstp = dst_hbm.at[p].at[c].at[s]
        pltpu.sync_copy(zeros_hbm, zero_v)

        def gath(j, b):
            return pltpu.make_async_copy(tab_hbm.at[src_v.at[j]],
                                         bufs[b], gsem[b])

        def scat(j, b):
            return pltpu.make_async_copy(bufs[b],
                                         acc.at[dst_v.at[j]], ssem[b])

        def phase(plane, outp):
            def zc(k, carry):
                pltpu.make_async_copy(
                    zero_v, acc.at[pl.ds(s * ROWS_PT + k * ZROWS, ZROWS)],
                    zsem).start()
                return carry
            lax.fori_loop(0, ZCH, zc, 0)

            def zw(k, carry):
                pltpu.make_async_copy(
                    zero_v, acc.at[pl.ds(s * ROWS_PT + k * ZROWS, ZROWS)],
                    zsem).wait()
                return carry
            lax.fori_loop(0, ZCH, zw, 0)
            plsc.subcore_barrier()
            srcp = src_hbm.at[plane].at[c].at[s]

            def block(b, carry):
                pltpu.sync_copy(srcp.at[pl.ds(b * BCH, BCH)], src_v)
                pltpu.sync_copy(dstp.at[pl.ds(b * BCH, BCH)], dst_v)
                gath(0, 0).start()
                gath(1, 1).start()

                def quad(q, carry2):
                    for k in range(4):
                        j = 4 * q + k
                        gath(j, k).wait()
                        scat(j, k).start(add=True)

                        @pl.when(j >= 2)
                        def _():
                            scat(j - 2, (k + 2) % 4).wait()

                        @pl.when(j + 2 < BCH)
                        def _():
                            gath(j + 2, (k + 2) % 4).start()
                    return carry2
                lax.fori_loop(0, BCH // 4, quad, 0)
                scat(BCH - 2, 2).wait()
                scat(BCH - 1, 3).wait()
                return carry
            lax.fori_loop(0, NBLK, block, 0)
            plsc.subcore_barrier()
            pltpu.sync_copy(acc.at[pl.ds(s * ROWS_PT, ROWS_PT)],
                            outp.at[c].at[pl.ds(s * ROWS_PT, ROWS_PT)])
            plsc.subcore_barrier()

        plane_a = jnp.where(p > 0, 2 * p - 1, 0)
        phase(plane_a, out_hbm.at[0])

        @pl.when(p > 0)
        def _():
            phase(2 * p, out_hbm.at[1])

    return conv_kernel(tab, srcs, dsts, zeros_h)


def _row_spec(w):
    return pl.BlockSpec((ROWT, w), lambda i: (i, 0))


def _full_spec(shape):
    nd = len(shape)
    return pl.BlockSpec(shape, lambda i: (0,) * nd)


def _tc_encoder(context, target, mf,
                W1, b1, g1, be1, W2, b2, Wt, bt, mask_token, Wg1, Wtg1):
    def body(ctx_r, tgt_r, mf_r, W1r, b1r, g1r, be1r,
             W2r, b2r, Wtr, btr, mtokr, Wg1r, Wtg1r,
             fused_r, ha_r, ht_r):
        x = jnp.dot(ctx_r[...], W1r[...],
                    preferred_element_type=jnp.float32) + b1r[...]
        x = _gelu(_ln(x, g1r[...], be1r[...]))
        x = _gelu(jnp.dot(x, W2r[...],
                          preferred_element_type=jnp.float32) + b2r[...])
        m = mf_r[...]
        mt = tgt_r[...] * (1.0 - m) + mtokr[...] * m
        t = _gelu(jnp.dot(mt, Wtr[...],
                          preferred_element_type=jnp.float32) + btr[...])
        fused = jnp.concatenate([x, t], axis=1)
        fused_r[...] = fused
        ha_r[...] = jnp.dot(fused, Wg1r[...],
                            preferred_element_type=jnp.float32)
        ht_r[...] = jnp.dot(fused, Wtg1r[...],
                            preferred_element_type=jnp.float32)

    return pl.pallas_call(
        body,
        grid=(GRID,),
        in_specs=[
            _row_spec(CTX), _row_spec(TGT), _row_spec(TGT),
            _full_spec((CTX, H)), _full_spec((1, H)), _full_spec((1, H)),
            _full_spec((1, H)), _full_spec((H, H)), _full_spec((1, H)),
            _full_spec((TGT, HH)), _full_spec((1, HH)), _full_spec((1, TGT)),
            _full_spec((H + HH, H)), _full_spec((H + HH, H)),
        ],
        out_specs=[_row_spec(H + HH), _row_spec(H), _row_spec(H)],
        out_shape=[
            jax.ShapeDtypeStruct((N, H + HH), jnp.float32),
            jax.ShapeDtypeStruct((N, H), jnp.float32),
            jax.ShapeDtypeStruct((N, H), jnp.float32),
        ],
    )(context, target, mf, W1, b1, g1, be1, W2, b2, Wt, bt,
      mask_token, Wg1, Wtg1)


def _tc_scale(dega, degt, ha, ht):
    def body(dega_r, degt_r, ha_r, ht_r, ga_r, gt_r):
        da = lax.rsqrt(dega_r[...][:, :1] + 1.0)
        dt = lax.rsqrt(degt_r[...][:, :1] + 1.0)
        ga_r[...] = ha_r[...] * da
        gt_r[...] = ht_r[...] * dt

    return pl.pallas_call(
        body,
        grid=(GRID,),
        in_specs=[_row_spec(DW), _row_spec(DW), _row_spec(H), _row_spec(H)],
        out_specs=[_row_spec(H), _row_spec(H)],
        out_shape=[
            jax.ShapeDtypeStruct((N, H), jnp.float32),
            jax.ShapeDtypeStruct((N, H), jnp.float32),
        ],
    )(dega, degt, ha, ht)


def _tc_mid(S, g, deg, bg, W):
    def body(S_r, g_r, deg_r, bgr, Wr, out_r):
        d = lax.rsqrt(deg_r[...][:, :1] + 1.0)
        h = _gelu(d * (S_r[...] + g_r[...]) + bgr[...])
        out_r[...] = jnp.dot(h, Wr[...],
                             preferred_element_type=jnp.float32) * d

    return pl.pallas_call(
        body,
        grid=(GRID,),
        in_specs=[
            _row_spec(H), _row_spec(H), _row_spec(DW),
            _full_spec((1, H)), _full_spec((H, H)),
        ],
        out_specs=_row_spec(H),
        out_shape=jax.ShapeDtypeStruct((N, H), jnp.float32),
    )(S, g, deg, bg, W)


def _tc_head(S2a, S2t, g2a, g2t, dega, degt, fused, alpha,
             bg2, btg2, Wh1, bh1, gh, beh, Wh2, bh2):
    def body(S2a_r, S2t_r, g2a_r, g2t_r, dega_r, degt_r, fused_r, al_r,
             bg2r, btg2r, Wh1r, bh1r, ghr, behr, Wh2r, bh2r, out_r):
        da = lax.rsqrt(dega_r[...][:, :1] + 1.0)
        dt = lax.rsqrt(degt_r[...][:, :1] + 1.0)
        h_sp = da * (S2a_r[...] + g2a_r[...]) + bg2r[...]
        h_tr = dt * (S2t_r[...] + g2t_r[...]) + btg2r[...]
        a = jax.nn.sigmoid(al_r[...])
        h = a * h_sp + (1.0 - a) * h_tr
        z = jnp.concatenate([h, fused_r[...]], axis=1)
        z = jnp.dot(z, Wh1r[...], preferred_element_type=jnp.float32)
        z = _gelu(_ln(z + bh1r[...], ghr[...], behr[...]))
        out_r[...] = jnp.dot(z, Wh2r[...],
                             preferred_element_type=jnp.float32) + bh2r[...]

    D2 = H + H + HH
    return pl.pallas_call(
        body,
        grid=(GRID,),
        in_specs=[
            _row_spec(H), _row_spec(H), _row_spec(H), _row_spec(H),
            _row_spec(DW), _row_spec(DW), _row_spec(H + HH),
            _full_spec((1, 1)),
            _full_spec((1, H)), _full_spec((1, H)),
            _full_spec((D2, H)), _full_spec((1, H)),
            _full_spec((1, H)), _full_spec((1, H)),
            _full_spec((H, TGT)), _full_spec((1, TGT)),
        ],
        out_specs=_row_spec(TGT),
        out_shape=jax.ShapeDtypeStruct((N, TGT), jnp.float32),
    )(S2a, S2t, g2a, g2t, dega, degt, fused, alpha,
      bg2, btg2, Wh1, bh1, gh, beh, Wh2, bh2)


def _tile_layout(idx):
    return idx.reshape(NS, NCHUNK, CH)


def _prep_edges(ei):
    src = ei[0].astype(jnp.int32)
    dst = ei[1].astype(jnp.int32)
    pad = E_PAD - E
    srcp = jnp.concatenate([src, jnp.zeros((pad,), jnp.int32)])
    dstp = jnp.concatenate([dst, jnp.full((pad,), JUNK, jnp.int32)])
    return srcp, dstp


def _quarters(g):
    return jnp.concatenate(
        [g[:, 0:QW], g[:, QW:2 * QW], g[:, 2 * QW:3 * QW], g[:, 3 * QW:4 * QW]],
        axis=0)


def _merge4(S):
    return jnp.concatenate(
        [S[0, 0, :N], S[0, 1, :N], S[1, 0, :N], S[1, 1, :N]], axis=1)


def kernel(context, target, mask, adj_ei, transit_ei, W1, b1, g1, be1, W2, b2,
           Wt, bt, mask_token, Wg1, bg1, Wg2, bg2, Wtg1, btg1, Wtg2, btg2,
           alpha, Wh1, bh1, gh, beh, Wh2, bh2):
    f32 = jnp.float32
    mf = mask.astype(f32)
    r = lambda v: v.reshape(1, -1).astype(f32)

    src_a, dst_a = _prep_edges(adj_ei)
    src_t, dst_t = _prep_edges(transit_ei)

    la, lt = _tile_layout(dst_a), _tile_layout(dst_t)
    dd = jnp.stack([jnp.stack([la, lt])])
    da = jnp.stack([la, la])[None]
    dt = jnp.stack([lt, lt])[None]
    dsts_xs = jnp.concatenate([dd, da, dt, da, dt])

    def offs(src, p):
        return jnp.stack([_tile_layout(src + (2 * p) * N),
                          _tile_layout(src + (2 * p + 1) * N)])

    sa_lo, sa_hi = offs(src_a, 0)[None], offs(src_a, 1)[None]
    st_lo, st_hi = offs(src_t, 0)[None], offs(src_t, 1)[None]
    sd = jnp.stack([jnp.stack([_tile_layout(dst_a),
                               _tile_layout(dst_t + N)])])

    srcs_xs = jnp.concatenate(
        [sd, sa_lo, sa_hi, st_lo, st_hi, sa_lo, sa_hi, st_lo, st_hi])

    zeros_c = jnp.zeros((ZROWS, QW), f32)
    fused, ha, ht = _tc_encoder(
        context, target, mf, W1, r(b1), r(g1), r(be1), W2, r(b2), Wt, r(bt),
        mask_token.reshape(1, TGT), Wg1, Wtg1)

    zN = jnp.zeros((N, H), f32)
    init = dict(
        next_tab=jnp.ones((4 * N, QW), f32),
        pend_tab=jnp.zeros((4 * N, QW), f32),
        ga=zN, gt=zN, g2a=zN, g2t=zN, S2a=zN, S2t=zN,
        dega=jnp.ones((N, DW), f32), degt=jnp.ones((N, DW), f32),
    )

    def b_deg(cr, S):
        dega = S[0, 0, :N, :DW]
        degt = S[0, 1, :N, :DW]
        ga, gt = _tc_scale(dega, degt, ha, ht)
        return dict(cr, next_tab=_quarters(ga), pend_tab=_quarters(gt),
                    ga=ga, gt=gt, dega=dega, degt=degt)

    def b_c1a(cr, S):
        g2a = _tc_mid(_merge4(S), cr["ga"], cr["dega"], r(bg1), Wg2)
        return dict(cr, next_tab=cr["pend_tab"], pend_tab=_quarters(g2a),
                    g2a=g2a)

    def b_c1t(cr, S):
        g2t = _tc_mid(_merge4(S), cr["gt"], cr["degt"], r(btg1), Wtg2)
        return dict(cr, next_tab=cr["pend_tab"], pend_tab=_quarters(g2t),
                    g2t=g2t)

    def b_c2a(cr, S):
        return dict(cr, next_tab=cr["pend_tab"], S2a=_merge4(S))

    def b_c2t(cr, S):
        return dict(cr, S2t=_merge4(S))

    n_pass = lax.optimization_barrier(jnp.int32(5))

    def cond(st):
        i, _ = st
        return i < n_pass

    def body(st):
        i, cr = st
        tag = jnp.full((8, QW), 1.0, f32) * i.astype(f32)
        tab = jnp.concatenate([cr["next_tab"], tag], axis=0)
        S = _sc_conv(tab, srcs_xs, dsts_xs, zeros_c)
        cr = lax.switch(i, [b_deg, b_c1a, b_c1t, b_c2a, b_c2t], cr, S)
        return (i + 1, cr)

    _, fin = lax.while_loop(cond, body, (jnp.int32(0), init))

    return _tc_head(fin["S2a"], fin["S2t"], fin["g2a"], fin["g2t"],
                    fin["dega"], fin["degt"], fused, alpha.reshape(1, 1),
                    r(bg2), r(btg2), Wh1, r(bh1), r(gh), r(beh), Wh2, r(bh2))


# speedup vs baseline: 9.8826x; 1.0110x over previous
import functools
import math

import jax
import jax.numpy as jnp
from jax import lax
from jax.experimental import pallas as pl
from jax.experimental.pallas import tpu as pltpu
from jax.experimental.pallas import tpu_sc as plsc

N = 50000
E = 800000
H = 64
QW = 16
HH = 32
CTX = 128
TGT = 16

NC = 2
NS = 16
CH = 128
NCHUNK = 392
NBLK = 2
BCH = NCHUNK // NBLK
EPT = NCHUNK * CH
E_PAD = EPT * NS
ACC_ROWS = 50176
ROWS_PT = ACC_ROWS // NS
ZROWS = 112
ZCH = ROWS_PT // ZROWS
DW = 8
JUNK = N

ROWT = 2000
GRID = N // ROWT


def _gelu(x):
    return 0.5 * x * (1.0 + lax.erf(x * (1.0 / math.sqrt(2.0))))


def _ln(x, g, b):
    m = jnp.mean(x, axis=-1, keepdims=True)
    v = jnp.mean((x - m) ** 2, axis=-1, keepdims=True)
    return (x - m) * lax.rsqrt(v + 1e-5) * g + b


def _sc_mesh():
    return plsc.VectorSubcoreMesh(core_axis_name="c", subcore_axis_name="s",
                                  num_cores=NC, num_subcores=NS)


def _sc_conv(tab, srcs, dsts, zeros_h):
    @functools.partial(
        pl.kernel,
        out_type=jax.ShapeDtypeStruct((4, NC, ACC_ROWS, QW), jnp.float32),
        mesh=_sc_mesh(),
        scratch_types=[
            pltpu.VMEM((BCH, CH), jnp.int32),
            pltpu.VMEM((BCH, CH), jnp.int32),
            pltpu.VMEM((CH, QW), jnp.float32),
            pltpu.VMEM((CH, QW), jnp.float32),
            pltpu.VMEM((CH, QW), jnp.float32),
            pltpu.VMEM((CH, QW), jnp.float32),
            pltpu.VMEM((ZROWS, QW), jnp.float32),
            pltpu.VMEM((8, QW), jnp.float32),
            pltpu.VMEM_SHARED((ACC_ROWS, QW), jnp.float32),
            pltpu.SemaphoreType.DMA,
            pltpu.SemaphoreType.DMA,
            pltpu.SemaphoreType.DMA,
            pltpu.SemaphoreType.DMA,
            pltpu.SemaphoreType.DMA,
            pltpu.SemaphoreType.DMA,
            pltpu.SemaphoreType.DMA,
            pltpu.SemaphoreType.DMA,
            pltpu.SemaphoreType.DMA,
        ],
        compiler_params=pltpu.CompilerParams(use_tc_tiling_on_sc=False,
                                             needs_layout_passes=False),
    )
    def conv_kernel(tab_hbm, src_hbm, dst_hbm, zeros_hbm, out_hbm,
                    src_v, dst_v, buf0, buf1, buf2, buf3, zero_v, tag_v, acc,
                    g0, g1, g2, g3, s0, s1, s2, s3, zsem):
        c = lax.axis_index("c")
        s = lax.axis_index("s")
        bufs = [buf0, buf1, buf2, buf3]
        gsem = [g0, g1, g2, g3]
        ssem = [s0, s1, s2, s3]
        pltpu.sync_copy(tab_hbm.at[pl.ds(8 * N, 8)], tag_v)
        p = jnp.max(tag_v[0, :]).astype(jnp.int32)
        pltpu.sync_copy(zeros_hbm, zero_v)

        def gath(j, b):
            return pltpu.make_async_copy(tab_hbm.at[src_v.at[j]],
                                         bufs[b], gsem[b])

        def scat(j, b):
            return pltpu.make_async_copy(bufs[b],
                                         acc.at[dst_v.at[j]], ssem[b])

        def phase(plane, pair, outp):
            def zc(k, carry):
                pltpu.make_async_copy(
                    zero_v, acc.at[pl.ds(s * ROWS_PT + k * ZROWS, ZROWS)],
                    zsem).start()
                return carry
            lax.fori_loop(0, ZCH, zc, 0)

            def zw(k, carry):
                pltpu.make_async_copy(
                    zero_v, acc.at[pl.ds(s * ROWS_PT + k * ZROWS, ZROWS)],
                    zsem).wait()
                return carry
            lax.fori_loop(0, ZCH, zw, 0)
            plsc.subcore_barrier()
            srcp = src_hbm.at[plane].at[c].at[s]
            dstp = dst_hbm.at[p].at[pair].at[c].at[s]

            def block(b, carry):
                pltpu.sync_copy(srcp.at[pl.ds(b * BCH, BCH)], src_v)
                pltpu.sync_copy(dstp.at[pl.ds(b * BCH, BCH)], dst_v)
                gath(0, 0).start()
                gath(1, 1).start()

                def quad(q, carry2):
                    for k in range(4):
                        j = 4 * q + k
                        gath(j, k).wait()
                        scat(j, k).start(add=True)

                        @pl.when(j >= 2)
                        def _():
                            scat(j - 2, (k + 2) % 4).wait()

                        @pl.when(j + 2 < BCH)
                        def _():
                            gath(j + 2, (k + 2) % 4).start()
                    return carry2
                lax.fori_loop(0, BCH // 4, quad, 0)
                scat(BCH - 2, 2).wait()
                scat(BCH - 1, 3).wait()
                return carry
            lax.fori_loop(0, NBLK, block, 0)
            plsc.subcore_barrier()
            pltpu.sync_copy(acc.at[pl.ds(s * ROWS_PT, ROWS_PT)],
                            outp.at[c].at[pl.ds(s * ROWS_PT, ROWS_PT)])
            plsc.subcore_barrier()

        plane0 = jnp.where(p > 0, 1, 0)
        phase(plane0, 0, out_hbm.at[0])

        @pl.when(p > 0)
        def _():
            phase(2, 0, out_hbm.at[1])
            phase(3, 1, out_hbm.at[2])
            phase(4, 1, out_hbm.at[3])

    return conv_kernel(tab, srcs, dsts, zeros_h)


def _row_spec(w):
    return pl.BlockSpec((ROWT, w), lambda i: (i, 0))


def _full_spec(shape):
    nd = len(shape)
    return pl.BlockSpec(shape, lambda i: (0,) * nd)


def _tc_encoder(context, target, mf,
                W1, b1, g1, be1, W2, b2, Wt, bt, mask_token, Wg1, Wtg1):
    def body(ctx_r, tgt_r, mf_r, W1r, b1r, g1r, be1r,
             W2r, b2r, Wtr, btr, mtokr, Wg1r, Wtg1r,
             fused_r, ha_r, ht_r):
        x = jnp.dot(ctx_r[...], W1r[...],
                    preferred_element_type=jnp.float32) + b1r[...]
        x = _gelu(_ln(x, g1r[...], be1r[...]))
        x = _gelu(jnp.dot(x, W2r[...],
                          preferred_element_type=jnp.float32) + b2r[...])
        m = mf_r[...]
        mt = tgt_r[...] * (1.0 - m) + mtokr[...] * m
        t = _gelu(jnp.dot(mt, Wtr[...],
                          preferred_element_type=jnp.float32) + btr[...])
        fused = jnp.concatenate([x, t], axis=1)
        fused_r[...] = fused
        ha_r[...] = jnp.dot(fused, Wg1r[...],
                            preferred_element_type=jnp.float32)
        ht_r[...] = jnp.dot(fused, Wtg1r[...],
                            preferred_element_type=jnp.float32)

    return pl.pallas_call(
        body,
        grid=(GRID,),
        in_specs=[
            _row_spec(CTX), _row_spec(TGT), _row_spec(TGT),
            _full_spec((CTX, H)), _full_spec((1, H)), _full_spec((1, H)),
            _full_spec((1, H)), _full_spec((H, H)), _full_spec((1, H)),
            _full_spec((TGT, HH)), _full_spec((1, HH)), _full_spec((1, TGT)),
            _full_spec((H + HH, H)), _full_spec((H + HH, H)),
        ],
        out_specs=[_row_spec(H + HH), _row_spec(H), _row_spec(H)],
        out_shape=[
            jax.ShapeDtypeStruct((N, H + HH), jnp.float32),
            jax.ShapeDtypeStruct((N, H), jnp.float32),
            jax.ShapeDtypeStruct((N, H), jnp.float32),
        ],
    )(context, target, mf, W1, b1, g1, be1, W2, b2, Wt, bt,
      mask_token, Wg1, Wtg1)


def _tc_scale(dega, degt, ha, ht):
    def body(dega_r, degt_r, ha_r, ht_r, ga_r, gt_r):
        da = lax.rsqrt(dega_r[...][:, :1] + 1.0)
        dt = lax.rsqrt(degt_r[...][:, :1] + 1.0)
        ga_r[...] = ha_r[...] * da
        gt_r[...] = ht_r[...] * dt

    return pl.pallas_call(
        body,
        grid=(GRID,),
        in_specs=[_row_spec(DW), _row_spec(DW), _row_spec(H), _row_spec(H)],
        out_specs=[_row_spec(H), _row_spec(H)],
        out_shape=[
            jax.ShapeDtypeStruct((N, H), jnp.float32),
            jax.ShapeDtypeStruct((N, H), jnp.float32),
        ],
    )(dega, degt, ha, ht)


def _tc_mid(S, g, deg, bg, W):
    def body(S_r, g_r, deg_r, bgr, Wr, out_r):
        d = lax.rsqrt(deg_r[...][:, :1] + 1.0)
        h = _gelu(d * (S_r[...] + g_r[...]) + bgr[...])
        out_r[...] = jnp.dot(h, Wr[...],
                             preferred_element_type=jnp.float32) * d

    return pl.pallas_call(
        body,
        grid=(GRID,),
        in_specs=[
            _row_spec(H), _row_spec(H), _row_spec(DW),
            _full_spec((1, H)), _full_spec((H, H)),
        ],
        out_specs=_row_spec(H),
        out_shape=jax.ShapeDtypeStruct((N, H), jnp.float32),
    )(S, g, deg, bg, W)


def _tc_head(S2a, S2t, g2a, g2t, dega, degt, fused, alpha,
             bg2, btg2, Wh1, bh1, gh, beh, Wh2, bh2):
    def body(S2a_r, S2t_r, g2a_r, g2t_r, dega_r, degt_r, fused_r, al_r,
             bg2r, btg2r, Wh1r, bh1r, ghr, behr, Wh2r, bh2r, out_r):
        da = lax.rsqrt(dega_r[...][:, :1] + 1.0)
        dt = lax.rsqrt(degt_r[...][:, :1] + 1.0)
        h_sp = da * (S2a_r[...] + g2a_r[...]) + bg2r[...]
        h_tr = dt * (S2t_r[...] + g2t_r[...]) + btg2r[...]
        a = jax.nn.sigmoid(al_r[...])
        h = a * h_sp + (1.0 - a) * h_tr
        z = jnp.concatenate([h, fused_r[...]], axis=1)
        z = jnp.dot(z, Wh1r[...], preferred_element_type=jnp.float32)
        z = _gelu(_ln(z + bh1r[...], ghr[...], behr[...]))
        out_r[...] = jnp.dot(z, Wh2r[...],
                             preferred_element_type=jnp.float32) + bh2r[...]

    D2 = H + H + HH
    return pl.pallas_call(
        body,
        grid=(GRID,),
        in_specs=[
            _row_spec(H), _row_spec(H), _row_spec(H), _row_spec(H),
            _row_spec(DW), _row_spec(DW), _row_spec(H + HH),
            _full_spec((1, 1)),
            _full_spec((1, H)), _full_spec((1, H)),
            _full_spec((D2, H)), _full_spec((1, H)),
            _full_spec((1, H)), _full_spec((1, H)),
            _full_spec((H, TGT)), _full_spec((1, TGT)),
        ],
        out_specs=_row_spec(TGT),
        out_shape=jax.ShapeDtypeStruct((N, TGT), jnp.float32),
    )(S2a, S2t, g2a, g2t, dega, degt, fused, alpha,
      bg2, btg2, Wh1, bh1, gh, beh, Wh2, bh2)


def _tile_layout(idx):
    return idx.reshape(NS, NCHUNK, CH)


def _prep_edges(ei):
    src = ei[0].astype(jnp.int32)
    dst = ei[1].astype(jnp.int32)
    pad = E_PAD - E
    srcp = jnp.concatenate([src, jnp.zeros((pad,), jnp.int32)])
    dstp = jnp.concatenate([dst, jnp.full((pad,), JUNK, jnp.int32)])
    return srcp, dstp


def _quarters(g):
    return jnp.concatenate(
        [g[:, 0:QW], g[:, QW:2 * QW], g[:, 2 * QW:3 * QW], g[:, 3 * QW:4 * QW]],
        axis=0)


def _merge4(S):
    return jnp.concatenate(
        [S[0, 0, :N], S[0, 1, :N], S[1, 0, :N], S[1, 1, :N]], axis=1)


def kernel(context, target, mask, adj_ei, transit_ei, W1, b1, g1, be1, W2, b2,
           Wt, bt, mask_token, Wg1, bg1, Wg2, bg2, Wtg1, btg1, Wtg2, btg2,
           alpha, Wh1, bh1, gh, beh, Wh2, bh2):
    f32 = jnp.float32
    mf = mask.astype(f32)
    r = lambda v: v.reshape(1, -1).astype(f32)

    src_a, dst_a = _prep_edges(adj_ei)
    src_t, dst_t = _prep_edges(transit_ei)

    la, lt = _tile_layout(dst_a), _tile_layout(dst_t)
    dsts_xs = jnp.stack([
        jnp.stack([jnp.stack([la, lt]), jnp.stack([la, lt])]),
        jnp.stack([jnp.stack([la, la]), jnp.stack([lt, lt])]),
        jnp.stack([jnp.stack([la, la]), jnp.stack([lt, lt])]),
    ])

    def two(srcp, o0, o1):
        return jnp.stack([_tile_layout(srcp + o0 * N),
                          _tile_layout(srcp + o1 * N)])

    srcs_xs = jnp.stack([
        jnp.stack([_tile_layout(dst_a), _tile_layout(dst_t + N)]),
        two(src_a, 0, 1), two(src_a, 2, 3),
        two(src_t, 4, 5), two(src_t, 6, 7),
    ])

    zeros_c = jnp.zeros((ZROWS, QW), f32)
    fused, ha, ht = _tc_encoder(
        context, target, mf, W1, r(b1), r(g1), r(be1), W2, r(b2), Wt, r(bt),
        mask_token.reshape(1, TGT), Wg1, Wtg1)

    def tab8(xa, xt):
        return jnp.concatenate([_quarters(xa), _quarters(xt)], axis=0)

    def halfS(S, q0, q1):
        return jnp.concatenate(
            [S[q0, 0, :N], S[q0, 1, :N], S[q1, 0, :N], S[q1, 1, :N]], axis=1)

    zN = jnp.zeros((N, H), f32)
    init = dict(
        next_tab=jnp.ones((8 * N, QW), f32),
        ga=zN, gt=zN, g2a=zN, g2t=zN, S2a=zN, S2t=zN,
        dega=jnp.ones((N, DW), f32), degt=jnp.ones((N, DW), f32),
    )

    def b_deg(cr, S):
        dega = S[0, 0, :N, :DW]
        degt = S[0, 1, :N, :DW]
        ga, gt = _tc_scale(dega, degt, ha, ht)
        return dict(cr, next_tab=tab8(ga, gt),
                    ga=ga, gt=gt, dega=dega, degt=degt)

    def b_l1(cr, S):
        g2a = _tc_mid(halfS(S, 0, 1), cr["ga"], cr["dega"], r(bg1), Wg2)
        g2t = _tc_mid(halfS(S, 2, 3), cr["gt"], cr["degt"], r(btg1), Wtg2)
        return dict(cr, next_tab=tab8(g2a, g2t), g2a=g2a, g2t=g2t)

    def b_l2(cr, S):
        return dict(cr, S2a=halfS(S, 0, 1), S2t=halfS(S, 2, 3))

    n_pass = lax.optimization_barrier(jnp.int32(3))

    def cond(st):
        i, _ = st
        return i < n_pass

    def body(st):
        i, cr = st
        tag = jnp.full((8, QW), 1.0, f32) * i.astype(f32)
        tab = jnp.concatenate([cr["next_tab"], tag], axis=0)
        S = _sc_conv(tab, srcs_xs, dsts_xs, zeros_c)
        cr = lax.switch(i, [b_deg, b_l1, b_l2], cr, S)
        return (i + 1, cr)

    _, fin = lax.while_loop(cond, body, (jnp.int32(0), init))

    return _tc_head(fin["S2a"], fin["S2t"], fin["g2a"], fin["g2t"],
                    fin["dega"], fin["degt"], fused, alpha.reshape(1, 1),
                    r(bg2), r(btg2), Wh1, r(bh1), r(gh), r(beh), Wh2, r(bh2))


# speedup vs baseline: 10.7539x; 1.0882x over previous
import functools
import math

import jax
import jax.numpy as jnp
from jax import lax
from jax.experimental import pallas as pl
from jax.experimental.pallas import tpu as pltpu
from jax.experimental.pallas import tpu_sc as plsc

N = 50000
E = 800000
H = 64
QW = 16
HH = 32
CTX = 128
TGT = 16

NC = 2
NS = 16
CH = 128
NCHUNK = 392
NBLK = 2
BCH = NCHUNK // NBLK
EPT = NCHUNK * CH
E_PAD = EPT * NS
ACC_ROWS = 50176
ROWS_PT = ACC_ROWS // NS
ZROWS = 112
ZCH = ROWS_PT // ZROWS
DW = 8
JUNK = N

ROWT = 2000
GRID = N // ROWT


def _gelu(x):
    return 0.5 * x * (1.0 + lax.erf(x * (1.0 / math.sqrt(2.0))))


def _ln(x, g, b):
    m = jnp.mean(x, axis=-1, keepdims=True)
    v = jnp.mean((x - m) ** 2, axis=-1, keepdims=True)
    return (x - m) * lax.rsqrt(v + 1e-5) * g + b


def _sc_mesh():
    return plsc.VectorSubcoreMesh(core_axis_name="c", subcore_axis_name="s",
                                  num_cores=NC, num_subcores=NS)


def _sc_conv(tab, srcs, dsts, zeros_h):
    @functools.partial(
        pl.kernel,
        out_type=jax.ShapeDtypeStruct((4, NC, ACC_ROWS, QW), jnp.float32),
        mesh=_sc_mesh(),
        scratch_types=[
            pltpu.VMEM((BCH, CH), jnp.int32),
            pltpu.VMEM((BCH, CH), jnp.int32),
            pltpu.VMEM((CH, QW), jnp.float32),
            pltpu.VMEM((CH, QW), jnp.float32),
            pltpu.VMEM((CH, QW), jnp.float32),
            pltpu.VMEM((CH, QW), jnp.float32),
            pltpu.VMEM((CH, QW), jnp.float32),
            pltpu.VMEM((CH, QW), jnp.float32),
            pltpu.VMEM((CH, QW), jnp.float32),
            pltpu.VMEM((ZROWS, QW), jnp.float32),
            pltpu.VMEM((8, QW), jnp.float32),
            pltpu.VMEM_SHARED((ACC_ROWS, QW), jnp.float32),
        ] + [pltpu.SemaphoreType.DMA] * 15,
        compiler_params=pltpu.CompilerParams(use_tc_tiling_on_sc=False,
                                             needs_layout_passes=False),
    )
    def conv_kernel(tab_hbm, src_hbm, dst_hbm, zeros_hbm, out_hbm,
                    src_v, dst_v, buf0, buf1, buf2, buf3, buf4, buf5, buf6,
                    zero_v, tag_v, acc,
                    g0, g1, g2, g3, g4, g5, g6,
                    s0, s1, s2, s3, s4, s5, s6, zsem):
        c = lax.axis_index("c")
        s = lax.axis_index("s")
        bufs = [buf0, buf1, buf2, buf3, buf4, buf5, buf6]
        gsem = [g0, g1, g2, g3, g4, g5, g6]
        ssem = [s0, s1, s2, s3, s4, s5, s6]
        pltpu.sync_copy(tab_hbm.at[pl.ds(8 * N, 8)], tag_v)
        p = jnp.max(tag_v[0, :]).astype(jnp.int32)
        pltpu.sync_copy(zeros_hbm, zero_v)

        def gath(j, b):
            return pltpu.make_async_copy(tab_hbm.at[src_v.at[j]],
                                         bufs[b], gsem[b])

        def scat(j, b):
            return pltpu.make_async_copy(bufs[b],
                                         acc.at[dst_v.at[j]], ssem[b])

        def phase(plane, pair, outp):
            def zc(k, carry):
                pltpu.make_async_copy(
                    zero_v, acc.at[pl.ds(s * ROWS_PT + k * ZROWS, ZROWS)],
                    zsem).start()
                return carry
            lax.fori_loop(0, ZCH, zc, 0)

            def zw(k, carry):
                pltpu.make_async_copy(
                    zero_v, acc.at[pl.ds(s * ROWS_PT + k * ZROWS, ZROWS)],
                    zsem).wait()
                return carry
            lax.fori_loop(0, ZCH, zw, 0)
            plsc.subcore_barrier()
            srcp = src_hbm.at[plane].at[c].at[s]
            dstp = dst_hbm.at[p].at[pair].at[c].at[s]

            def block(b, carry):
                pltpu.sync_copy(srcp.at[pl.ds(b * BCH, BCH)], src_v)
                pltpu.sync_copy(dstp.at[pl.ds(b * BCH, BCH)], dst_v)
                gath(0, 0).start()
                gath(1, 1).start()
                gath(2, 2).start()

                def sept(q, carry2):
                    for k in range(7):
                        j = 7 * q + k
                        gath(j, k).wait()
                        scat(j, k).start(add=True)

                        @pl.when(j >= 4)
                        def _():
                            scat(j - 4, (k + 3) % 7).wait()

                        @pl.when(j + 3 < BCH)
                        def _():
                            gath(j + 3, (k + 3) % 7).start()
                    return carry2
                lax.fori_loop(0, BCH // 7, sept, 0)
                scat(BCH - 4, (BCH - 4) % 7).wait()
                scat(BCH - 3, (BCH - 3) % 7).wait()
                scat(BCH - 2, (BCH - 2) % 7).wait()
                scat(BCH - 1, (BCH - 1) % 7).wait()
                return carry
            lax.fori_loop(0, NBLK, block, 0)
            plsc.subcore_barrier()
            pltpu.sync_copy(acc.at[pl.ds(s * ROWS_PT, ROWS_PT)],
                            outp.at[c].at[pl.ds(s * ROWS_PT, ROWS_PT)])
            plsc.subcore_barrier()

        plane0 = jnp.where(p > 0, 1, 0)
        phase(plane0, 0, out_hbm.at[0])

        @pl.when(p > 0)
        def _():
            phase(2, 0, out_hbm.at[1])
            phase(3, 1, out_hbm.at[2])
            phase(4, 1, out_hbm.at[3])

    return conv_kernel(tab, srcs, dsts, zeros_h)


def _row_spec(w):
    return pl.BlockSpec((ROWT, w), lambda i: (i, 0))


def _full_spec(shape):
    nd = len(shape)
    return pl.BlockSpec(shape, lambda i: (0,) * nd)


def _tc_encoder(context, target, mf,
                W1, b1, g1, be1, W2, b2, Wt, bt, mask_token, Wg1, Wtg1):
    def body(ctx_r, tgt_r, mf_r, W1r, b1r, g1r, be1r,
             W2r, b2r, Wtr, btr, mtokr, Wg1r, Wtg1r,
             fused_r, ha_r, ht_r):
        x = jnp.dot(ctx_r[...], W1r[...],
                    preferred_element_type=jnp.float32) + b1r[...]
        x = _gelu(_ln(x, g1r[...], be1r[...]))
        x = _gelu(jnp.dot(x, W2r[...],
                          preferred_element_type=jnp.float32) + b2r[...])
        m = mf_r[...]
        mt = tgt_r[...] * (1.0 - m) + mtokr[...] * m
        t = _gelu(jnp.dot(mt, Wtr[...],
                          preferred_element_type=jnp.float32) + btr[...])
        fused = jnp.concatenate([x, t], axis=1)
        fused_r[...] = fused
        ha_r[...] = jnp.dot(fused, Wg1r[...],
                            preferred_element_type=jnp.float32)
        ht_r[...] = jnp.dot(fused, Wtg1r[...],
                            preferred_element_type=jnp.float32)

    return pl.pallas_call(
        body,
        grid=(GRID,),
        in_specs=[
            _row_spec(CTX), _row_spec(TGT), _row_spec(TGT),
            _full_spec((CTX, H)), _full_spec((1, H)), _full_spec((1, H)),
            _full_spec((1, H)), _full_spec((H, H)), _full_spec((1, H)),
            _full_spec((TGT, HH)), _full_spec((1, HH)), _full_spec((1, TGT)),
            _full_spec((H + HH, H)), _full_spec((H + HH, H)),
        ],
        out_specs=[_row_spec(H + HH), _row_spec(H), _row_spec(H)],
        out_shape=[
            jax.ShapeDtypeStruct((N, H + HH), jnp.float32),
            jax.ShapeDtypeStruct((N, H), jnp.float32),
            jax.ShapeDtypeStruct((N, H), jnp.float32),
        ],
    )(context, target, mf, W1, b1, g1, be1, W2, b2, Wt, bt,
      mask_token, Wg1, Wtg1)


def _tc_scale(dega, degt, ha, ht):
    def body(dega_r, degt_r, ha_r, ht_r, ga_r, gt_r):
        da = lax.rsqrt(dega_r[...][:, :1] + 1.0)
        dt = lax.rsqrt(degt_r[...][:, :1] + 1.0)
        ga_r[...] = ha_r[...] * da
        gt_r[...] = ht_r[...] * dt

    return pl.pallas_call(
        body,
        grid=(GRID,),
        in_specs=[_row_spec(DW), _row_spec(DW), _row_spec(H), _row_spec(H)],
        out_specs=[_row_spec(H), _row_spec(H)],
        out_shape=[
            jax.ShapeDtypeStruct((N, H), jnp.float32),
            jax.ShapeDtypeStruct((N, H), jnp.float32),
        ],
    )(dega, degt, ha, ht)


def _tc_mid(S, g, deg, bg, W):
    def body(S_r, g_r, deg_r, bgr, Wr, out_r):
        d = lax.rsqrt(deg_r[...][:, :1] + 1.0)
        h = _gelu(d * (S_r[...] + g_r[...]) + bgr[...])
        out_r[...] = jnp.dot(h, Wr[...],
                             preferred_element_type=jnp.float32) * d

    return pl.pallas_call(
        body,
        grid=(GRID,),
        in_specs=[
            _row_spec(H), _row_spec(H), _row_spec(DW),
            _full_spec((1, H)), _full_spec((H, H)),
        ],
        out_specs=_row_spec(H),
        out_shape=jax.ShapeDtypeStruct((N, H), jnp.float32),
    )(S, g, deg, bg, W)


def _tc_head(S2a, S2t, g2a, g2t, dega, degt, fused, alpha,
             bg2, btg2, Wh1, bh1, gh, beh, Wh2, bh2):
    def body(S2a_r, S2t_r, g2a_r, g2t_r, dega_r, degt_r, fused_r, al_r,
             bg2r, btg2r, Wh1r, bh1r, ghr, behr, Wh2r, bh2r, out_r):
        da = lax.rsqrt(dega_r[...][:, :1] + 1.0)
        dt = lax.rsqrt(degt_r[...][:, :1] + 1.0)
        h_sp = da * (S2a_r[...] + g2a_r[...]) + bg2r[...]
        h_tr = dt * (S2t_r[...] + g2t_r[...]) + btg2r[...]
        a = jax.nn.sigmoid(al_r[...])
        h = a * h_sp + (1.0 - a) * h_tr
        z = jnp.concatenate([h, fused_r[...]], axis=1)
        z = jnp.dot(z, Wh1r[...], preferred_element_type=jnp.float32)
        z = _gelu(_ln(z + bh1r[...], ghr[...], behr[...]))
        out_r[...] = jnp.dot(z, Wh2r[...],
                             preferred_element_type=jnp.float32) + bh2r[...]

    D2 = H + H + HH
    return pl.pallas_call(
        body,
        grid=(GRID,),
        in_specs=[
            _row_spec(H), _row_spec(H), _row_spec(H), _row_spec(H),
            _row_spec(DW), _row_spec(DW), _row_spec(H + HH),
            _full_spec((1, 1)),
            _full_spec((1, H)), _full_spec((1, H)),
            _full_spec((D2, H)), _full_spec((1, H)),
            _full_spec((1, H)), _full_spec((1, H)),
            _full_spec((H, TGT)), _full_spec((1, TGT)),
        ],
        out_specs=_row_spec(TGT),
        out_shape=jax.ShapeDtypeStruct((N, TGT), jnp.float32),
    )(S2a, S2t, g2a, g2t, dega, degt, fused, alpha,
      bg2, btg2, Wh1, bh1, gh, beh, Wh2, bh2)


def _tile_layout(idx):
    return idx.reshape(NS, NCHUNK, CH)


def _prep_edges(ei):
    src = ei[0].astype(jnp.int32)
    dst = ei[1].astype(jnp.int32)
    pad = E_PAD - E
    srcp = jnp.concatenate([src, jnp.zeros((pad,), jnp.int32)])
    dstp = jnp.concatenate([dst, jnp.full((pad,), JUNK, jnp.int32)])
    return srcp, dstp


def _quarters(g):
    return jnp.concatenate(
        [g[:, 0:QW], g[:, QW:2 * QW], g[:, 2 * QW:3 * QW], g[:, 3 * QW:4 * QW]],
        axis=0)


def _merge4(S):
    return jnp.concatenate(
        [S[0, 0, :N], S[0, 1, :N], S[1, 0, :N], S[1, 1, :N]], axis=1)


def kernel(context, target, mask, adj_ei, transit_ei, W1, b1, g1, be1, W2, b2,
           Wt, bt, mask_token, Wg1, bg1, Wg2, bg2, Wtg1, btg1, Wtg2, btg2,
           alpha, Wh1, bh1, gh, beh, Wh2, bh2):
    f32 = jnp.float32
    mf = mask.astype(f32)
    r = lambda v: v.reshape(1, -1).astype(f32)

    src_a, dst_a = _prep_edges(adj_ei)
    src_t, dst_t = _prep_edges(transit_ei)

    la, lt = _tile_layout(dst_a), _tile_layout(dst_t)
    dsts_xs = jnp.stack([
        jnp.stack([jnp.stack([la, lt]), jnp.stack([la, lt])]),
        jnp.stack([jnp.stack([la, la]), jnp.stack([lt, lt])]),
        jnp.stack([jnp.stack([la, la]), jnp.stack([lt, lt])]),
    ])

    def two(srcp, o0, o1):
        return jnp.stack([_tile_layout(srcp + o0 * N),
                          _tile_layout(srcp + o1 * N)])

    srcs_xs = jnp.stack([
        jnp.stack([_tile_layout(dst_a), _tile_layout(dst_t + N)]),
        two(src_a, 0, 1), two(src_a, 2, 3),
        two(src_t, 4, 5), two(src_t, 6, 7),
    ])

    zeros_c = jnp.zeros((ZROWS, QW), f32)
    fused, ha, ht = _tc_encoder(
        context, target, mf, W1, r(b1), r(g1), r(be1), W2, r(b2), Wt, r(bt),
        mask_token.reshape(1, TGT), Wg1, Wtg1)

    def tab8(xa, xt):
        return jnp.concatenate([_quarters(xa), _quarters(xt)], axis=0)

    def halfS(S, q0, q1):
        return jnp.concatenate(
            [S[q0, 0, :N], S[q0, 1, :N], S[q1, 0, :N], S[q1, 1, :N]], axis=1)

    zN = jnp.zeros((N, H), f32)
    init = dict(
        next_tab=jnp.ones((8 * N, QW), f32),
        ga=zN, gt=zN, g2a=zN, g2t=zN, S2a=zN, S2t=zN,
        dega=jnp.ones((N, DW), f32), degt=jnp.ones((N, DW), f32),
    )

    def b_deg(cr, S):
        dega = S[0, 0, :N, :DW]
        degt = S[0, 1, :N, :DW]
        ga, gt = _tc_scale(dega, degt, ha, ht)
        return dict(cr, next_tab=tab8(ga, gt),
                    ga=ga, gt=gt, dega=dega, degt=degt)

    def b_l1(cr, S):
        g2a = _tc_mid(halfS(S, 0, 1), cr["ga"], cr["dega"], r(bg1), Wg2)
        g2t = _tc_mid(halfS(S, 2, 3), cr["gt"], cr["degt"], r(btg1), Wtg2)
        return dict(cr, next_tab=tab8(g2a, g2t), g2a=g2a, g2t=g2t)

    def b_l2(cr, S):
        return dict(cr, S2a=halfS(S, 0, 1), S2t=halfS(S, 2, 3))

    n_pass = lax.optimization_barrier(jnp.int32(3))

    def cond(st):
        i, _ = st
        return i < n_pass

    def body(st):
        i, cr = st
        tag = jnp.full((8, QW), 1.0, f32) * i.astype(f32)
        tab = jnp.concatenate([cr["next_tab"], tag], axis=0)
        S = _sc_conv(tab, srcs_xs, dsts_xs, zeros_c)
        cr = lax.switch(i, [b_deg, b_l1, b_l2], cr, S)
        return (i + 1, cr)

    _, fin = lax.while_loop(cond, body, (jnp.int32(0), init))

    return _tc_head(fin["S2a"], fin["S2t"], fin["g2a"], fin["g2t"],
                    fin["dega"], fin["degt"], fused, alpha.reshape(1, 1),
                    r(bg2), r(btg2), Wh1, r(bh1), r(gh), r(beh), Wh2, r(bh2))


# speedup vs baseline: 11.7440x; 1.0921x over previous
import functools
import math

import jax
import jax.numpy as jnp
from jax import lax
from jax.experimental import pallas as pl
from jax.experimental.pallas import tpu as pltpu
from jax.experimental.pallas import tpu_sc as plsc

N = 50000
E = 800000
H = 64
QW = 16
HH = 32
CTX = 128
TGT = 16

NC = 2
NS = 16
CH = 128
NCHUNK = 392
NBLK = 2
BCH = NCHUNK // NBLK
EPT = NCHUNK * CH
E_PAD = EPT * NS
ACC_ROWS = 50176
ROWS_PT = ACC_ROWS // NS
ZROWS = 112
ZCH = ROWS_PT // ZROWS
DW = 8
JUNK = N

ROWT = 2000
GRID = N // ROWT


def _gelu(x):
    return 0.5 * x * (1.0 + lax.erf(x * (1.0 / math.sqrt(2.0))))


def _ln(x, g, b):
    m = jnp.mean(x, axis=-1, keepdims=True)
    v = jnp.mean((x - m) ** 2, axis=-1, keepdims=True)
    return (x - m) * lax.rsqrt(v + 1e-5) * g + b


def _sc_mesh():
    return plsc.VectorSubcoreMesh(core_axis_name="c", subcore_axis_name="s",
                                  num_cores=NC, num_subcores=NS)


def _sc_conv(tab, srcs, dsts, zeros_h):
    @functools.partial(
        pl.kernel,
        out_type=jax.ShapeDtypeStruct((4, NC, ACC_ROWS, QW), jnp.float32),
        mesh=_sc_mesh(),
        scratch_types=[
            pltpu.VMEM((BCH, CH), jnp.int32),
            pltpu.VMEM((BCH, CH), jnp.int32),
        ] + [pltpu.VMEM((CH, QW), jnp.float32)] * 14 + [
            pltpu.VMEM((ZROWS, QW), jnp.float32),
            pltpu.VMEM((8, QW), jnp.float32),
            pltpu.VMEM_SHARED((ACC_ROWS, QW), jnp.float32),
        ] + [pltpu.SemaphoreType.DMA] * 29,
        compiler_params=pltpu.CompilerParams(use_tc_tiling_on_sc=False,
                                             needs_layout_passes=False),
    )
    def conv_kernel(tab_hbm, src_hbm, dst_hbm, zeros_hbm, out_hbm,
                    src_v, dst_v, *rest):
        bufs = list(rest[0:14])
        zero_v = rest[14]
        tag_v = rest[15]
        acc = rest[16]
        gsem = list(rest[17:31])
        ssem = list(rest[31:45])
        zsem = rest[45]
        c = lax.axis_index("c")
        s = lax.axis_index("s")
        pltpu.sync_copy(tab_hbm.at[pl.ds(8 * N, 8)], tag_v)
        p = jnp.max(tag_v[0, :]).astype(jnp.int32)
        pltpu.sync_copy(zeros_hbm, zero_v)

        def gath(j, b):
            return pltpu.make_async_copy(tab_hbm.at[src_v.at[j]],
                                         bufs[b], gsem[b])

        def scat(j, b):
            return pltpu.make_async_copy(bufs[b],
                                         acc.at[dst_v.at[j]], ssem[b])

        def phase(plane, pair, outp):
            def zc(k, carry):
                pltpu.make_async_copy(
                    zero_v, acc.at[pl.ds(s * ROWS_PT + k * ZROWS, ZROWS)],
                    zsem).start()
                return carry
            lax.fori_loop(0, ZCH, zc, 0)

            def zw(k, carry):
                pltpu.make_async_copy(
                    zero_v, acc.at[pl.ds(s * ROWS_PT + k * ZROWS, ZROWS)],
                    zsem).wait()
                return carry
            lax.fori_loop(0, ZCH, zw, 0)
            plsc.subcore_barrier()
            srcp = src_hbm.at[plane].at[c].at[s]
            dstp = dst_hbm.at[p].at[pair].at[c].at[s]

            def block(b, carry):
                pltpu.sync_copy(srcp.at[pl.ds(b * BCH, BCH)], src_v)
                pltpu.sync_copy(dstp.at[pl.ds(b * BCH, BCH)], dst_v)
                for g in range(6):
                    gath(g, g).start()

                def grp(q, carry2):
                    for k in range(14):
                        j = 14 * q + k
                        gath(j, k).wait()
                        scat(j, k).start(add=True)

                        @pl.when(j >= 8)
                        def _():
                            scat(j - 8, (k + 6) % 14).wait()

                        @pl.when(j + 6 < BCH)
                        def _():
                            gath(j + 6, (k + 6) % 14).start()
                    return carry2
                lax.fori_loop(0, BCH // 14, grp, 0)
                for t in range(8):
                    scat(BCH - 8 + t, (BCH - 8 + t) % 14).wait()
                return carry
            lax.fori_loop(0, NBLK, block, 0)
            plsc.subcore_barrier()
            pltpu.sync_copy(acc.at[pl.ds(s * ROWS_PT, ROWS_PT)],
                            outp.at[c].at[pl.ds(s * ROWS_PT, ROWS_PT)])
            plsc.subcore_barrier()

        plane0 = jnp.where(p > 0, 1, 0)
        phase(plane0, 0, out_hbm.at[0])

        @pl.when(p > 0)
        def _():
            phase(2, 0, out_hbm.at[1])
            phase(3, 1, out_hbm.at[2])
            phase(4, 1, out_hbm.at[3])

    return conv_kernel(tab, srcs, dsts, zeros_h)


def _row_spec(w):
    return pl.BlockSpec((ROWT, w), lambda i: (i, 0))


def _full_spec(shape):
    nd = len(shape)
    return pl.BlockSpec(shape, lambda i: (0,) * nd)


def _tc_encoder(context, target, mf,
                W1, b1, g1, be1, W2, b2, Wt, bt, mask_token, Wg1, Wtg1):
    def body(ctx_r, tgt_r, mf_r, W1r, b1r, g1r, be1r,
             W2r, b2r, Wtr, btr, mtokr, Wg1r, Wtg1r,
             fused_r, ha_r, ht_r):
        x = jnp.dot(ctx_r[...], W1r[...],
                    preferred_element_type=jnp.float32) + b1r[...]
        x = _gelu(_ln(x, g1r[...], be1r[...]))
        x = _gelu(jnp.dot(x, W2r[...],
                          preferred_element_type=jnp.float32) + b2r[...])
        m = mf_r[...]
        mt = tgt_r[...] * (1.0 - m) + mtokr[...] * m
        t = _gelu(jnp.dot(mt, Wtr[...],
                          preferred_element_type=jnp.float32) + btr[...])
        fused = jnp.concatenate([x, t], axis=1)
        fused_r[...] = fused
        ha_r[...] = jnp.dot(fused, Wg1r[...],
                            preferred_element_type=jnp.float32)
        ht_r[...] = jnp.dot(fused, Wtg1r[...],
                            preferred_element_type=jnp.float32)

    return pl.pallas_call(
        body,
        grid=(GRID,),
        in_specs=[
            _row_spec(CTX), _row_spec(TGT), _row_spec(TGT),
            _full_spec((CTX, H)), _full_spec((1, H)), _full_spec((1, H)),
            _full_spec((1, H)), _full_spec((H, H)), _full_spec((1, H)),
            _full_spec((TGT, HH)), _full_spec((1, HH)), _full_spec((1, TGT)),
            _full_spec((H + HH, H)), _full_spec((H + HH, H)),
        ],
        out_specs=[_row_spec(H + HH), _row_spec(H), _row_spec(H)],
        out_shape=[
            jax.ShapeDtypeStruct((N, H + HH), jnp.float32),
            jax.ShapeDtypeStruct((N, H), jnp.float32),
            jax.ShapeDtypeStruct((N, H), jnp.float32),
        ],
    )(context, target, mf, W1, b1, g1, be1, W2, b2, Wt, bt,
      mask_token, Wg1, Wtg1)


def _tc_scale(dega, degt, ha, ht):
    def body(dega_r, degt_r, ha_r, ht_r, ga_r, gt_r):
        da = lax.rsqrt(dega_r[...][:, :1] + 1.0)
        dt = lax.rsqrt(degt_r[...][:, :1] + 1.0)
        ga_r[...] = ha_r[...] * da
        gt_r[...] = ht_r[...] * dt

    return pl.pallas_call(
        body,
        grid=(GRID,),
        in_specs=[_row_spec(DW), _row_spec(DW), _row_spec(H), _row_spec(H)],
        out_specs=[_row_spec(H), _row_spec(H)],
        out_shape=[
            jax.ShapeDtypeStruct((N, H), jnp.float32),
            jax.ShapeDtypeStruct((N, H), jnp.float32),
        ],
    )(dega, degt, ha, ht)


def _tc_mid(S, g, deg, bg, W):
    def body(S_r, g_r, deg_r, bgr, Wr, out_r):
        d = lax.rsqrt(deg_r[...][:, :1] + 1.0)
        h = _gelu(d * (S_r[...] + g_r[...]) + bgr[...])
        out_r[...] = jnp.dot(h, Wr[...],
                             preferred_element_type=jnp.float32) * d

    return pl.pallas_call(
        body,
        grid=(GRID,),
        in_specs=[
            _row_spec(H), _row_spec(H), _row_spec(DW),
            _full_spec((1, H)), _full_spec((H, H)),
        ],
        out_specs=_row_spec(H),
        out_shape=jax.ShapeDtypeStruct((N, H), jnp.float32),
    )(S, g, deg, bg, W)


def _tc_head(S2a, S2t, g2a, g2t, dega, degt, fused, alpha,
             bg2, btg2, Wh1, bh1, gh, beh, Wh2, bh2):
    def body(S2a_r, S2t_r, g2a_r, g2t_r, dega_r, degt_r, fused_r, al_r,
             bg2r, btg2r, Wh1r, bh1r, ghr, behr, Wh2r, bh2r, out_r):
        da = lax.rsqrt(dega_r[...][:, :1] + 1.0)
        dt = lax.rsqrt(degt_r[...][:, :1] + 1.0)
        h_sp = da * (S2a_r[...] + g2a_r[...]) + bg2r[...]
        h_tr = dt * (S2t_r[...] + g2t_r[...]) + btg2r[...]
        a = jax.nn.sigmoid(al_r[...])
        h = a * h_sp + (1.0 - a) * h_tr
        z = jnp.concatenate([h, fused_r[...]], axis=1)
        z = jnp.dot(z, Wh1r[...], preferred_element_type=jnp.float32)
        z = _gelu(_ln(z + bh1r[...], ghr[...], behr[...]))
        out_r[...] = jnp.dot(z, Wh2r[...],
                             preferred_element_type=jnp.float32) + bh2r[...]

    D2 = H + H + HH
    return pl.pallas_call(
        body,
        grid=(GRID,),
        in_specs=[
            _row_spec(H), _row_spec(H), _row_spec(H), _row_spec(H),
            _row_spec(DW), _row_spec(DW), _row_spec(H + HH),
            _full_spec((1, 1)),
            _full_spec((1, H)), _full_spec((1, H)),
            _full_spec((D2, H)), _full_spec((1, H)),
            _full_spec((1, H)), _full_spec((1, H)),
            _full_spec((H, TGT)), _full_spec((1, TGT)),
        ],
        out_specs=_row_spec(TGT),
        out_shape=jax.ShapeDtypeStruct((N, TGT), jnp.float32),
    )(S2a, S2t, g2a, g2t, dega, degt, fused, alpha,
      bg2, btg2, Wh1, bh1, gh, beh, Wh2, bh2)


def _tile_layout(idx):
    return idx.reshape(NS, NCHUNK, CH)


def _prep_edges(ei):
    src = ei[0].astype(jnp.int32)
    dst = ei[1].astype(jnp.int32)
    pad = E_PAD - E
    srcp = jnp.concatenate([src, jnp.zeros((pad,), jnp.int32)])
    dstp = jnp.concatenate([dst, jnp.full((pad,), JUNK, jnp.int32)])
    return srcp, dstp


def _quarters(g):
    return jnp.concatenate(
        [g[:, 0:QW], g[:, QW:2 * QW], g[:, 2 * QW:3 * QW], g[:, 3 * QW:4 * QW]],
        axis=0)


def _merge4(S):
    return jnp.concatenate(
        [S[0, 0, :N], S[0, 1, :N], S[1, 0, :N], S[1, 1, :N]], axis=1)


def kernel(context, target, mask, adj_ei, transit_ei, W1, b1, g1, be1, W2, b2,
           Wt, bt, mask_token, Wg1, bg1, Wg2, bg2, Wtg1, btg1, Wtg2, btg2,
           alpha, Wh1, bh1, gh, beh, Wh2, bh2):
    f32 = jnp.float32
    mf = mask.astype(f32)
    r = lambda v: v.reshape(1, -1).astype(f32)

    src_a, dst_a = _prep_edges(adj_ei)
    src_t, dst_t = _prep_edges(transit_ei)

    la, lt = _tile_layout(dst_a), _tile_layout(dst_t)
    dsts_xs = jnp.stack([
        jnp.stack([jnp.stack([la, lt]), jnp.stack([la, lt])]),
        jnp.stack([jnp.stack([la, la]), jnp.stack([lt, lt])]),
        jnp.stack([jnp.stack([la, la]), jnp.stack([lt, lt])]),
    ])

    def two(srcp, o0, o1):
        return jnp.stack([_tile_layout(srcp + o0 * N),
                          _tile_layout(srcp + o1 * N)])

    srcs_xs = jnp.stack([
        jnp.stack([_tile_layout(dst_a), _tile_layout(dst_t + N)]),
        two(src_a, 0, 1), two(src_a, 2, 3),
        two(src_t, 4, 5), two(src_t, 6, 7),
    ])

    zeros_c = jnp.zeros((ZROWS, QW), f32)
    fused, ha, ht = _tc_encoder(
        context, target, mf, W1, r(b1), r(g1), r(be1), W2, r(b2), Wt, r(bt),
        mask_token.reshape(1, TGT), Wg1, Wtg1)

    def tab8(xa, xt):
        return jnp.concatenate([_quarters(xa), _quarters(xt)], axis=0)

    def halfS(S, q0, q1):
        return jnp.concatenate(
            [S[q0, 0, :N], S[q0, 1, :N], S[q1, 0, :N], S[q1, 1, :N]], axis=1)

    zN = jnp.zeros((N, H), f32)
    init = dict(
        next_tab=jnp.ones((8 * N, QW), f32),
        ga=zN, gt=zN, g2a=zN, g2t=zN, S2a=zN, S2t=zN,
        dega=jnp.ones((N, DW), f32), degt=jnp.ones((N, DW), f32),
    )

    def b_deg(cr, S):
        dega = S[0, 0, :N, :DW]
        degt = S[0, 1, :N, :DW]
        ga, gt = _tc_scale(dega, degt, ha, ht)
        return dict(cr, next_tab=tab8(ga, gt),
                    ga=ga, gt=gt, dega=dega, degt=degt)

    def b_l1(cr, S):
        g2a = _tc_mid(halfS(S, 0, 1), cr["ga"], cr["dega"], r(bg1), Wg2)
        g2t = _tc_mid(halfS(S, 2, 3), cr["gt"], cr["degt"], r(btg1), Wtg2)
        return dict(cr, next_tab=tab8(g2a, g2t), g2a=g2a, g2t=g2t)

    def b_l2(cr, S):
        return dict(cr, S2a=halfS(S, 0, 1), S2t=halfS(S, 2, 3))

    n_pass = lax.optimization_barrier(jnp.int32(3))

    def cond(st):
        i, _ = st
        return i < n_pass

    def body(st):
        i, cr = st
        tag = jnp.full((8, QW), 1.0, f32) * i.astype(f32)
        tab = jnp.concatenate([cr["next_tab"], tag], axis=0)
        S = _sc_conv(tab, srcs_xs, dsts_xs, zeros_c)
        cr = lax.switch(i, [b_deg, b_l1, b_l2], cr, S)
        return (i + 1, cr)

    _, fin = lax.while_loop(cond, body, (jnp.int32(0), init))

    return _tc_head(fin["S2a"], fin["S2t"], fin["g2a"], fin["g2t"],
                    fin["dega"], fin["degt"], fused, alpha.reshape(1, 1),
                    r(bg2), r(btg2), Wh1, r(bh1), r(gh), r(beh), Wh2, r(bh2))
